# baseline (speedup 1.0000x reference)
import jax
import jax.numpy as jnp
from jax.experimental import pallas as pl
from jax.experimental.pallas import tpu as pltpu

_STRIP = 8


def _dice_body(smooth_ref, inp_ref, tgt_ref, out_ref, acc_i_ref, acc_d_ref):
    n = pl.program_id(0)
    num_n = pl.num_programs(0)
    smooth = smooth_ref[0, 0]
    C = inp_ref.shape[1]
    H = inp_ref.shape[2]
    W = inp_ref.shape[3]

    one = jnp.float32(1.0)
    zero = jnp.float32(0.0)
    groups = [(0, 1), (2, 3), (4,)]
    for grp in groups:
        acc_i = {c: jnp.zeros((_STRIP, W), jnp.float32) for c in grp}
        acc_d = {c: jnp.zeros((_STRIP, W), jnp.float32) for c in grp}
        for s in range(0, H, _STRIP):
            tv = tgt_ref[0, pl.ds(s, _STRIP), :]
            for c in grp:
                xv = inp_ref[0, c, pl.ds(s, _STRIP), :]
                m = jnp.where(tv == c, one, zero)
                acc_i[c] = acc_i[c] + m * xv
                acc_d[c] = acc_d[c] + (xv + m)
        for c in grp:
            acc_i_ref[n, c, :] = jnp.sum(acc_i[c], axis=0)
            acc_d_ref[n, c, :] = jnp.sum(acc_d[c], axis=0)

    @pl.when(n == num_n - 1)
    def _fini():
        inter = jnp.sum(acc_i_ref[...], axis=-1)
        den = jnp.sum(acc_d_ref[...], axis=-1)
        ratios = (2.0 * inter + smooth) / (den + smooth)
        out_ref[0, 0] = 1.0 - jnp.sum(ratios) / (num_n * C)


def kernel(inputs, targets, smooth):
    N, C, H, W = inputs.shape
    t32 = targets.astype(jnp.int32)
    s = jnp.asarray(smooth, jnp.float32).reshape(1, 1)
    out = pl.pallas_call(
        _dice_body,
        grid=(N,),
        in_specs=[
            pl.BlockSpec(memory_space=pltpu.SMEM),
            pl.BlockSpec((1, C, H, W), lambda n: (n, 0, 0, 0)),
            pl.BlockSpec((1, H, W), lambda n: (n, 0, 0)),
        ],
        out_specs=pl.BlockSpec(memory_space=pltpu.SMEM),
        out_shape=jax.ShapeDtypeStruct((1, 1), jnp.float32),
        scratch_shapes=[
            pltpu.VMEM((N, C, W), jnp.float32),
            pltpu.VMEM((N, C, W), jnp.float32),
        ],
    )(s, inputs, t32)
    return out[0, 0]


# speedup vs baseline: 1.0002x; 1.0002x over previous
import jax
import jax.numpy as jnp
from jax.experimental import pallas as pl
from jax.experimental.pallas import tpu as pltpu

_STRIP = 8


def _dice_body(smooth_ref, inp_ref, tgt_ref, out_ref, acc_i_ref, acc_d_ref):
    n = pl.program_id(0)
    num_n = pl.num_programs(0)
    smooth = smooth_ref[0, 0]
    C = inp_ref.shape[1]
    H = inp_ref.shape[2]
    W = inp_ref.shape[3]

    one = jnp.float32(1.0)
    zero = jnp.float32(0.0)
    for c in range(C):
        acc_i = jnp.zeros((_STRIP, W), jnp.float32)
        acc_d = jnp.zeros((_STRIP, W), jnp.float32)
        for s in range(0, H, _STRIP):
            tv = tgt_ref[0, pl.ds(s, _STRIP), :]
            xv = inp_ref[0, c, pl.ds(s, _STRIP), :]
            m = jnp.where(tv == c, one, zero)
            acc_i = acc_i + m * xv
            acc_d = acc_d + (xv + m)
        acc_i_ref[n, c, :] = jnp.sum(acc_i, axis=0)
        acc_d_ref[n, c, :] = jnp.sum(acc_d, axis=0)

    @pl.when(n == num_n - 1)
    def _fini():
        inter = jnp.sum(acc_i_ref[...], axis=-1)
        den = jnp.sum(acc_d_ref[...], axis=-1)
        ratios = (2.0 * inter + smooth) / (den + smooth)
        out_ref[0, 0] = 1.0 - jnp.sum(ratios) / (num_n * C)


def kernel(inputs, targets, smooth):
    N, C, H, W = inputs.shape
    t32 = targets.astype(jnp.int32)
    s = jnp.asarray(smooth, jnp.float32).reshape(1, 1)
    out = pl.pallas_call(
        _dice_body,
        grid=(N,),
        in_specs=[
            pl.BlockSpec(memory_space=pltpu.SMEM),
            pl.BlockSpec((1, C, H, W), lambda n: (n, 0, 0, 0)),
            pl.BlockSpec((1, H, W), lambda n: (n, 0, 0)),
        ],
        out_specs=pl.BlockSpec(memory_space=pltpu.SMEM),
        out_shape=jax.ShapeDtypeStruct((1, 1), jnp.float32),
        scratch_shapes=[
            pltpu.VMEM((N, C, W), jnp.float32),
            pltpu.VMEM((N, C, W), jnp.float32),
        ],
    )(s, inputs, t32)
    return out[0, 0]
